# baseline (speedup 1.0000x reference)
import functools

import jax
import jax.numpy as jnp
from jax import lax
from jax.experimental import pallas as pl
from jax.experimental.pallas import tpu as pltpu
from jax.experimental.pallas import tpu_sc as plsc

N = 10000
E = 320000
D = 128
H = 64
C = 125
ZC = 80
NT = 16
EPT = E // NT
KPT = EPT // C
NW = 32
EPW = E // NW
KPW = EPW // C
RPT = 624
PB = 6
GA = 3
BM = 1000


def _mesh():
    return plsc.VectorSubcoreMesh(core_axis_name="c", subcore_axis_name="s")


DW = 16


def _sc_degree(dst3):
    @functools.partial(
        pl.kernel,
        mesh=_mesh(),
        out_type=jax.ShapeDtypeStruct((2 * N, DW), jnp.float32),
        compiler_params=pltpu.CompilerParams(
            use_tc_tiling_on_sc=False, skip_device_barrier=True),
        scratch_types=[
            pltpu.VMEM((KPW, C), jnp.int32),
            pltpu.VMEM((C, DW), jnp.float32),
            pltpu.VMEM((C, DW), jnp.float32),
            pltpu.VMEM_SHARED((N, DW), jnp.float32),
        ],
    )
    def deg_kernel(dst_hbm, out_hbm, dst_v, ones_v, zbuf, acc_sh):
        c = lax.axis_index("c")
        s = lax.axis_index("s")
        wid = s * 2 + c

        def fill(i, carry):
            ones_v[i, pl.ds(0, 16)] = jnp.full((16,), 1.0 / DW, jnp.float32)
            zbuf[i, pl.ds(0, 16)] = jnp.zeros((16,), jnp.float32)
            return carry

        lax.fori_loop(0, C, fill, 0)
        pltpu.sync_copy(dst_hbm.at[wid], dst_v)

        tbase = s * RPT
        for q in range(8):
            n = ZC if q < 7 else RPT - 7 * ZC
            pltpu.sync_copy(zbuf.at[pl.ds(0, n)],
                            acc_sh.at[pl.ds(tbase + q * ZC, n)])

        @pl.when(s == NT - 1)
        def _():
            pltpu.sync_copy(zbuf.at[pl.ds(0, 16)],
                            acc_sh.at[pl.ds(NT * RPT, 16)])

        plsc.subcore_barrier()

        def body(j, carry):
            pltpu.sync_copy(ones_v, acc_sh.at[dst_v.at[j]], add=True)
            return carry

        lax.fori_loop(0, KPW, body, 0)
        plsc.subcore_barrier()

        for q in range(8):
            n = ZC if q < 7 else RPT - 7 * ZC
            pltpu.sync_copy(acc_sh.at[pl.ds(tbase + q * ZC, n)],
                            ones_v.at[pl.ds(0, n)])
            pltpu.sync_copy(ones_v.at[pl.ds(0, n)],
                            out_hbm.at[pl.ds(c * N + tbase + q * ZC, n)])

        @pl.when(s == NT - 1)
        def _():
            pltpu.sync_copy(acc_sh.at[pl.ds(NT * RPT, 16)],
                            zbuf.at[pl.ds(0, 16)])
            pltpu.sync_copy(zbuf.at[pl.ds(0, 16)],
                            out_hbm.at[pl.ds(c * N + NT * RPT, 16)])

    return deg_kernel(dst3)


def _sc_mp(hp2, src4, dst3):
    @functools.partial(
        pl.kernel,
        mesh=_mesh(),
        out_type=jax.ShapeDtypeStruct((2 * N, H), jnp.float32),
        compiler_params=pltpu.CompilerParams(
            use_tc_tiling_on_sc=False, skip_device_barrier=True),
        scratch_types=[
            pltpu.VMEM((KPT, C), jnp.int32),
            pltpu.VMEM((KPT, C), jnp.int32),
            pltpu.VMEM((PB, C, H), jnp.float32),
            pltpu.VMEM_SHARED((N, H), jnp.float32),
            [pltpu.SemaphoreType.DMA] * PB,
            [pltpu.SemaphoreType.DMA] * PB,
        ],
    )
    def mp_kernel(hp_hbm, src_hbm, dst_hbm, out_hbm,
                  src_v, dst_v, gbuf, acc_sh, gsems, ssems):
        c = lax.axis_index("c")
        s = lax.axis_index("s")

        pltpu.sync_copy(src_hbm.at[c, s], src_v)
        pltpu.sync_copy(dst_hbm.at[s], dst_v)

        def zrow(i, carry):
            for k in range(H // 16):
                gbuf[0, i, pl.ds(k * 16, 16)] = jnp.zeros((16,), jnp.float32)
            return carry

        lax.fori_loop(0, ZC, zrow, 0)
        tbase = s * RPT
        for q in range(8):
            n = ZC if q < 7 else RPT - 7 * ZC
            pltpu.sync_copy(gbuf.at[0].at[pl.ds(0, n)],
                            acc_sh.at[pl.ds(tbase + q * ZC, n)])

        @pl.when(s == NT - 1)
        def _():
            pltpu.sync_copy(gbuf.at[0].at[pl.ds(0, 16)],
                            acc_sh.at[pl.ds(NT * RPT, 16)])

        plsc.subcore_barrier()

        def gstart(j, b):
            pltpu.async_copy(hp_hbm.at[src_v.at[j]], gbuf.at[b], gsems[b])

        def gwait(j, b):
            pltpu.make_async_copy(hp_hbm.at[src_v.at[j]], gbuf.at[b],
                                  gsems[b]).wait()

        def sstart(j, b):
            pltpu.async_copy(gbuf.at[b], acc_sh.at[dst_v.at[j]], ssems[b],
                             add=True)

        def swait(j, b):
            pltpu.make_async_copy(gbuf.at[b], acc_sh.at[dst_v.at[j]],
                                  ssems[b]).wait()

        for j in range(GA):
            gstart(j, j)
        for j in range(GA):
            gwait(j, j)
            sstart(j, j)
            gstart(j + GA, (j + GA) % PB)

        NK = (KPT - GA - PB + 1) // PB

        def body(k, carry):
            j0 = PB * k + GA
            for t in range(PB):
                j = j0 + t
                b = (GA + t) % PB
                bb = (2 * GA + t) % PB
                gwait(j, b)
                sstart(j, b)
                swait(j + GA - PB, bb)
                gstart(j + GA, bb)
            return carry

        lax.fori_loop(0, NK, body, 0)
        for j in range(PB * NK + GA, KPT):
            b = j % PB
            bb = (j + GA) % PB
            gwait(j, b)
            sstart(j, b)
            swait(j + GA - PB, bb)
            if j + GA < KPT:
                gstart(j + GA, bb)
        for j in range(KPT - (PB - GA), KPT):
            swait(j, j % PB)

        plsc.subcore_barrier()

        for q in range(8):
            n = ZC if q < 7 else RPT - 7 * ZC
            pltpu.sync_copy(acc_sh.at[pl.ds(tbase + q * ZC, n)],
                            gbuf.at[0].at[pl.ds(0, n)])
            pltpu.sync_copy(gbuf.at[0].at[pl.ds(0, n)],
                            out_hbm.at[pl.ds(c * N + tbase + q * ZC, n)])

        @pl.when(s == NT - 1)
        def _():
            pltpu.sync_copy(acc_sh.at[pl.ds(NT * RPT, 16)],
                            gbuf.at[1].at[pl.ds(0, 16)])
            pltpu.sync_copy(gbuf.at[1].at[pl.ds(0, 16)],
                            out_hbm.at[pl.ds(c * N + NT * RPT, 16)])

    return mp_kernel(hp2, src4, dst3)


def _tc_first(degp3, x, Wsp):
    def body(deg_ref, x_ref, w_ref, o_ref):
        deg = (jnp.sum(deg_ref[0], axis=1, keepdims=True)
               + jnp.sum(deg_ref[1], axis=1, keepdims=True) + 1.0)
        dis = lax.rsqrt(deg)
        h = jnp.dot(x_ref[...], w_ref[0], preferred_element_type=jnp.float32)
        o_ref[0] = h * dis

    return pl.pallas_call(
        body,
        grid=(2, N // BM),
        in_specs=[
            pl.BlockSpec((2, BM, DW), lambda h, i: (0, i, 0)),
            pl.BlockSpec((BM, D), lambda h, i: (i, 0)),
            pl.BlockSpec((1, D, H), lambda h, i: (h, 0, 0)),
        ],
        out_specs=pl.BlockSpec((1, BM, H), lambda h, i: (h, i, 0)),
        out_shape=jax.ShapeDtypeStruct((2, N, H), jnp.float32),
    )(degp3, x, Wsp)


def _tc_mid(degp3, acc, hp, Wq, bsp):
    def body(deg_ref, a_ref, hp_ref, w_ref, b_ref, o_ref):
        deg = (jnp.sum(deg_ref[0], axis=1, keepdims=True)
               + jnp.sum(deg_ref[1], axis=1, keepdims=True) + 1.0)
        dis = lax.rsqrt(deg)
        x0 = jnp.maximum((a_ref[0] + hp_ref[0]) * dis + b_ref[0], 0.0)
        x1 = jnp.maximum((a_ref[1] + hp_ref[1]) * dis + b_ref[1], 0.0)
        h = (jnp.dot(x0, w_ref[0, 0], preferred_element_type=jnp.float32)
             + jnp.dot(x1, w_ref[1, 0], preferred_element_type=jnp.float32))
        o_ref[0] = h * dis

    return pl.pallas_call(
        body,
        grid=(2, N // BM),
        in_specs=[
            pl.BlockSpec((2, BM, DW), lambda h, i: (0, i, 0)),
            pl.BlockSpec((2, BM, H), lambda h, i: (0, i, 0)),
            pl.BlockSpec((2, BM, H), lambda h, i: (0, i, 0)),
            pl.BlockSpec((2, 1, H, H), lambda h, i: (0, h, 0, 0)),
            pl.BlockSpec((2, 1, H), lambda h, i: (0, 0, 0)),
        ],
        out_specs=pl.BlockSpec((1, BM, H), lambda h, i: (h, i, 0)),
        out_shape=jax.ShapeDtypeStruct((2, N, H), jnp.float32),
    )(degp3, acc, hp, Wq, bsp)


def _tc_last(degp3, acc, hp, b):
    def body(deg_ref, a_ref, hp_ref, b_ref, o_ref):
        deg = (jnp.sum(deg_ref[0], axis=1, keepdims=True)
               + jnp.sum(deg_ref[1], axis=1, keepdims=True) + 1.0)
        dis = lax.rsqrt(deg)
        y0 = (a_ref[0] + hp_ref[0]) * dis
        y1 = (a_ref[1] + hp_ref[1]) * dis
        o_ref[...] = jnp.concatenate([y0, y1], axis=1) + b_ref[...]

    return pl.pallas_call(
        body,
        grid=(N // BM,),
        in_specs=[
            pl.BlockSpec((2, BM, DW), lambda i: (0, i, 0)),
            pl.BlockSpec((2, BM, H), lambda i: (0, i, 0)),
            pl.BlockSpec((2, BM, H), lambda i: (0, i, 0)),
            pl.BlockSpec((1, D), lambda i: (0, 0)),
        ],
        out_specs=pl.BlockSpec((BM, D), lambda i: (i, 0)),
        out_shape=jax.ShapeDtypeStruct((N, D), jnp.float32),
    )(degp3, acc, hp, b)


def kernel(x, adj_t, W1, b1, W2, b2, W3, b3):
    adj = adj_t.astype(jnp.int32)
    src3 = adj[0].reshape(NT, KPT, C)
    src4 = jnp.stack([src3, src3 + N])
    dst3 = adj[1].reshape(NT, KPT, C)
    dst3d = adj[1].reshape(NW, KPW, C)

    degp = _sc_degree(dst3d)
    degp3 = degp.reshape(2, N, DW)

    def wq(W):
        return W.reshape(2, H, 2, H).transpose(0, 2, 1, 3)

    def wsp(W):
        return W.reshape(D, 2, H).transpose(1, 0, 2)

    hp1 = _tc_first(degp3, x, wsp(W1))
    acc1 = _sc_mp(hp1.reshape(2 * N, H), src4, dst3).reshape(2, N, H)
    hp2 = _tc_mid(degp3, acc1, hp1, wq(W2), b1.reshape(2, 1, H))
    acc2 = _sc_mp(hp2.reshape(2 * N, H), src4, dst3).reshape(2, N, H)
    hp3 = _tc_mid(degp3, acc2, hp2, wq(W3), b2.reshape(2, 1, H))
    acc3 = _sc_mp(hp3.reshape(2 * N, H), src4, dst3).reshape(2, N, H)
    return _tc_last(degp3, acc3, hp3, b3.reshape(1, D))


# speedup vs baseline: 1.0340x; 1.0340x over previous
import functools

import jax
import jax.numpy as jnp
from jax import lax
from jax.experimental import pallas as pl
from jax.experimental.pallas import tpu as pltpu
from jax.experimental.pallas import tpu_sc as plsc

N = 10000
E = 320000
D = 128
H = 64
C = 125
ZC = 80
NT = 16
EPT = E // NT
KPT = EPT // C
NW = 32
EPW = E // NW
KPW = EPW // C
RPT = 624
PB = 6
GA = 3
BM = 2000


def _mesh():
    return plsc.VectorSubcoreMesh(core_axis_name="c", subcore_axis_name="s")


DW = 16


def _sc_degree(dst3):
    @functools.partial(
        pl.kernel,
        mesh=_mesh(),
        out_type=jax.ShapeDtypeStruct((2 * N, DW), jnp.float32),
        compiler_params=pltpu.CompilerParams(
            use_tc_tiling_on_sc=False, skip_device_barrier=True),
        scratch_types=[
            pltpu.VMEM((KPW, C), jnp.int32),
            pltpu.VMEM((C, DW), jnp.float32),
            pltpu.VMEM((C, DW), jnp.float32),
            pltpu.VMEM_SHARED((N, DW), jnp.float32),
        ],
    )
    def deg_kernel(dst_hbm, out_hbm, dst_v, ones_v, zbuf, acc_sh):
        c = lax.axis_index("c")
        s = lax.axis_index("s")
        wid = s * 2 + c

        def fill(i, carry):
            ones_v[i, pl.ds(0, 16)] = jnp.full((16,), 1.0 / DW, jnp.float32)
            zbuf[i, pl.ds(0, 16)] = jnp.zeros((16,), jnp.float32)
            return carry

        lax.fori_loop(0, C, fill, 0)
        pltpu.sync_copy(dst_hbm.at[wid], dst_v)

        tbase = s * RPT
        for q in range(8):
            n = ZC if q < 7 else RPT - 7 * ZC
            pltpu.sync_copy(zbuf.at[pl.ds(0, n)],
                            acc_sh.at[pl.ds(tbase + q * ZC, n)])

        @pl.when(s == NT - 1)
        def _():
            pltpu.sync_copy(zbuf.at[pl.ds(0, 16)],
                            acc_sh.at[pl.ds(NT * RPT, 16)])

        plsc.subcore_barrier()

        def body(j, carry):
            pltpu.sync_copy(ones_v, acc_sh.at[dst_v.at[j]], add=True)
            return carry

        lax.fori_loop(0, KPW, body, 0)
        plsc.subcore_barrier()

        for q in range(8):
            n = ZC if q < 7 else RPT - 7 * ZC
            pltpu.sync_copy(acc_sh.at[pl.ds(tbase + q * ZC, n)],
                            ones_v.at[pl.ds(0, n)])
            pltpu.sync_copy(ones_v.at[pl.ds(0, n)],
                            out_hbm.at[pl.ds(c * N + tbase + q * ZC, n)])

        @pl.when(s == NT - 1)
        def _():
            pltpu.sync_copy(acc_sh.at[pl.ds(NT * RPT, 16)],
                            zbuf.at[pl.ds(0, 16)])
            pltpu.sync_copy(zbuf.at[pl.ds(0, 16)],
                            out_hbm.at[pl.ds(c * N + NT * RPT, 16)])

    return deg_kernel(dst3)


def _sc_mp(hp2, src4, dst3):
    @functools.partial(
        pl.kernel,
        mesh=_mesh(),
        out_type=jax.ShapeDtypeStruct((2 * N, H), jnp.float32),
        compiler_params=pltpu.CompilerParams(
            use_tc_tiling_on_sc=False, skip_device_barrier=True),
        scratch_types=[
            pltpu.VMEM((KPT, C), jnp.int32),
            pltpu.VMEM((KPT, C), jnp.int32),
            pltpu.VMEM((PB, C, H), jnp.float32),
            pltpu.VMEM_SHARED((N, H), jnp.float32),
            [pltpu.SemaphoreType.DMA] * PB,
            [pltpu.SemaphoreType.DMA] * PB,
        ],
    )
    def mp_kernel(hp_hbm, src_hbm, dst_hbm, out_hbm,
                  src_v, dst_v, gbuf, acc_sh, gsems, ssems):
        c = lax.axis_index("c")
        s = lax.axis_index("s")

        pltpu.sync_copy(src_hbm.at[c, s], src_v)
        pltpu.sync_copy(dst_hbm.at[s], dst_v)

        def zrow(i, carry):
            for k in range(H // 16):
                gbuf[0, i, pl.ds(k * 16, 16)] = jnp.zeros((16,), jnp.float32)
            return carry

        lax.fori_loop(0, ZC, zrow, 0)
        tbase = s * RPT
        for q in range(8):
            n = ZC if q < 7 else RPT - 7 * ZC
            pltpu.sync_copy(gbuf.at[0].at[pl.ds(0, n)],
                            acc_sh.at[pl.ds(tbase + q * ZC, n)])

        @pl.when(s == NT - 1)
        def _():
            pltpu.sync_copy(gbuf.at[0].at[pl.ds(0, 16)],
                            acc_sh.at[pl.ds(NT * RPT, 16)])

        plsc.subcore_barrier()

        def gstart(j, b):
            pltpu.async_copy(hp_hbm.at[src_v.at[j]], gbuf.at[b], gsems[b])

        def gwait(j, b):
            pltpu.make_async_copy(hp_hbm.at[src_v.at[j]], gbuf.at[b],
                                  gsems[b]).wait()

        def sstart(j, b):
            pltpu.async_copy(gbuf.at[b], acc_sh.at[dst_v.at[j]], ssems[b],
                             add=True)

        def swait(j, b):
            pltpu.make_async_copy(gbuf.at[b], acc_sh.at[dst_v.at[j]],
                                  ssems[b]).wait()

        for j in range(GA):
            gstart(j, j)
        for j in range(GA):
            gwait(j, j)
            sstart(j, j)
            gstart(j + GA, (j + GA) % PB)

        NK = (KPT - GA - PB + 1) // PB

        def body(k, carry):
            j0 = PB * k + GA
            for t in range(PB):
                j = j0 + t
                b = (GA + t) % PB
                bb = (2 * GA + t) % PB
                gwait(j, b)
                sstart(j, b)
                swait(j + GA - PB, bb)
                gstart(j + GA, bb)
            return carry

        lax.fori_loop(0, NK, body, 0)
        for j in range(PB * NK + GA, KPT):
            b = j % PB
            bb = (j + GA) % PB
            gwait(j, b)
            sstart(j, b)
            swait(j + GA - PB, bb)
            if j + GA < KPT:
                gstart(j + GA, bb)
        for j in range(KPT - (PB - GA), KPT):
            swait(j, j % PB)

        plsc.subcore_barrier()

        for q in range(8):
            n = ZC if q < 7 else RPT - 7 * ZC
            pltpu.sync_copy(acc_sh.at[pl.ds(tbase + q * ZC, n)],
                            gbuf.at[0].at[pl.ds(0, n)])
            pltpu.sync_copy(gbuf.at[0].at[pl.ds(0, n)],
                            out_hbm.at[pl.ds(c * N + tbase + q * ZC, n)])

        @pl.when(s == NT - 1)
        def _():
            pltpu.sync_copy(acc_sh.at[pl.ds(NT * RPT, 16)],
                            gbuf.at[1].at[pl.ds(0, 16)])
            pltpu.sync_copy(gbuf.at[1].at[pl.ds(0, 16)],
                            out_hbm.at[pl.ds(c * N + NT * RPT, 16)])

    return mp_kernel(hp2, src4, dst3)


def _tc_first(degp3, x, Wsp):
    def body(deg_ref, x_ref, w_ref, o_ref):
        deg = (jnp.sum(deg_ref[0], axis=1, keepdims=True)
               + jnp.sum(deg_ref[1], axis=1, keepdims=True) + 1.0)
        dis = lax.rsqrt(deg)
        h = jnp.dot(x_ref[...], w_ref[0], preferred_element_type=jnp.float32)
        o_ref[0] = h * dis

    return pl.pallas_call(
        body,
        grid=(2, N // BM),
        in_specs=[
            pl.BlockSpec((2, BM, DW), lambda h, i: (0, i, 0)),
            pl.BlockSpec((BM, D), lambda h, i: (i, 0)),
            pl.BlockSpec((1, D, H), lambda h, i: (h, 0, 0)),
        ],
        out_specs=pl.BlockSpec((1, BM, H), lambda h, i: (h, i, 0)),
        out_shape=jax.ShapeDtypeStruct((2, N, H), jnp.float32),
    )(degp3, x, Wsp)


def _tc_mid(degp3, acc, hp, Wq, bsp):
    def body(deg_ref, a_ref, hp_ref, w_ref, b_ref, o_ref):
        deg = (jnp.sum(deg_ref[0], axis=1, keepdims=True)
               + jnp.sum(deg_ref[1], axis=1, keepdims=True) + 1.0)
        dis = lax.rsqrt(deg)
        x0 = jnp.maximum((a_ref[0] + hp_ref[0]) * dis + b_ref[0], 0.0)
        x1 = jnp.maximum((a_ref[1] + hp_ref[1]) * dis + b_ref[1], 0.0)
        h = (jnp.dot(x0, w_ref[0, 0], preferred_element_type=jnp.float32)
             + jnp.dot(x1, w_ref[1, 0], preferred_element_type=jnp.float32))
        o_ref[0] = h * dis

    return pl.pallas_call(
        body,
        grid=(2, N // BM),
        in_specs=[
            pl.BlockSpec((2, BM, DW), lambda h, i: (0, i, 0)),
            pl.BlockSpec((2, BM, H), lambda h, i: (0, i, 0)),
            pl.BlockSpec((2, BM, H), lambda h, i: (0, i, 0)),
            pl.BlockSpec((2, 1, H, H), lambda h, i: (0, h, 0, 0)),
            pl.BlockSpec((2, 1, H), lambda h, i: (0, 0, 0)),
        ],
        out_specs=pl.BlockSpec((1, BM, H), lambda h, i: (h, i, 0)),
        out_shape=jax.ShapeDtypeStruct((2, N, H), jnp.float32),
    )(degp3, acc, hp, Wq, bsp)


def _tc_last(degp3, acc, hp, b):
    def body(deg_ref, a_ref, hp_ref, b_ref, o_ref):
        deg = (jnp.sum(deg_ref[0], axis=1, keepdims=True)
               + jnp.sum(deg_ref[1], axis=1, keepdims=True) + 1.0)
        dis = lax.rsqrt(deg)
        y0 = (a_ref[0] + hp_ref[0]) * dis
        y1 = (a_ref[1] + hp_ref[1]) * dis
        o_ref[...] = jnp.concatenate([y0, y1], axis=1) + b_ref[...]

    return pl.pallas_call(
        body,
        grid=(N // BM,),
        in_specs=[
            pl.BlockSpec((2, BM, DW), lambda i: (0, i, 0)),
            pl.BlockSpec((2, BM, H), lambda i: (0, i, 0)),
            pl.BlockSpec((2, BM, H), lambda i: (0, i, 0)),
            pl.BlockSpec((1, D), lambda i: (0, 0)),
        ],
        out_specs=pl.BlockSpec((BM, D), lambda i: (i, 0)),
        out_shape=jax.ShapeDtypeStruct((N, D), jnp.float32),
    )(degp3, acc, hp, b)


def kernel(x, adj_t, W1, b1, W2, b2, W3, b3):
    adj = adj_t.astype(jnp.int32)
    src3 = adj[0].reshape(NT, KPT, C)
    src4 = jnp.stack([src3, src3 + N])
    dst3 = adj[1].reshape(NT, KPT, C)
    dst3d = adj[1].reshape(NW, KPW, C)

    degp = _sc_degree(dst3d)
    degp3 = degp.reshape(2, N, DW)

    def wq(W):
        return W.reshape(2, H, 2, H).transpose(0, 2, 1, 3)

    def wsp(W):
        return W.reshape(D, 2, H).transpose(1, 0, 2)

    hp1 = _tc_first(degp3, x, wsp(W1))
    acc1 = _sc_mp(hp1.reshape(2 * N, H), src4, dst3).reshape(2, N, H)
    hp2 = _tc_mid(degp3, acc1, hp1, wq(W2), b1.reshape(2, 1, H))
    acc2 = _sc_mp(hp2.reshape(2 * N, H), src4, dst3).reshape(2, N, H)
    hp3 = _tc_mid(degp3, acc2, hp2, wq(W3), b2.reshape(2, 1, H))
    acc3 = _sc_mp(hp3.reshape(2 * N, H), src4, dst3).reshape(2, N, H)
    return _tc_last(degp3, acc3, hp3, b3.reshape(1, D))


# speedup vs baseline: 1.0548x; 1.0201x over previous
import functools

import jax
import jax.numpy as jnp
from jax import lax
from jax.experimental import pallas as pl
from jax.experimental.pallas import tpu as pltpu
from jax.experimental.pallas import tpu_sc as plsc

N = 10000
E = 320000
D = 128
H = 64
C = 125
ZC = 80
NT = 16
EPT = E // NT
KPT = EPT // C
NW = 32
EPW = E // NW
KPW = EPW // C
RPT = 624
PB = 6
GA = 3
BM = 10000


def _mesh():
    return plsc.VectorSubcoreMesh(core_axis_name="c", subcore_axis_name="s")


DW = 16


def _sc_degree(dst3):
    @functools.partial(
        pl.kernel,
        mesh=_mesh(),
        out_type=jax.ShapeDtypeStruct((2 * N, DW), jnp.float32),
        compiler_params=pltpu.CompilerParams(
            use_tc_tiling_on_sc=False, skip_device_barrier=True),
        scratch_types=[
            pltpu.VMEM((KPW, C), jnp.int32),
            pltpu.VMEM((C, DW), jnp.float32),
            pltpu.VMEM((C, DW), jnp.float32),
            pltpu.VMEM_SHARED((N, DW), jnp.float32),
        ],
    )
    def deg_kernel(dst_hbm, out_hbm, dst_v, ones_v, zbuf, acc_sh):
        c = lax.axis_index("c")
        s = lax.axis_index("s")
        wid = s * 2 + c

        def fill(i, carry):
            ones_v[i, pl.ds(0, 16)] = jnp.full((16,), 1.0 / DW, jnp.float32)
            zbuf[i, pl.ds(0, 16)] = jnp.zeros((16,), jnp.float32)
            return carry

        lax.fori_loop(0, C, fill, 0)
        pltpu.sync_copy(dst_hbm.at[wid], dst_v)

        tbase = s * RPT
        for q in range(8):
            n = ZC if q < 7 else RPT - 7 * ZC
            pltpu.sync_copy(zbuf.at[pl.ds(0, n)],
                            acc_sh.at[pl.ds(tbase + q * ZC, n)])

        @pl.when(s == NT - 1)
        def _():
            pltpu.sync_copy(zbuf.at[pl.ds(0, 16)],
                            acc_sh.at[pl.ds(NT * RPT, 16)])

        plsc.subcore_barrier()

        def body(j, carry):
            pltpu.sync_copy(ones_v, acc_sh.at[dst_v.at[j]], add=True)
            return carry

        lax.fori_loop(0, KPW, body, 0)
        plsc.subcore_barrier()

        for q in range(8):
            n = ZC if q < 7 else RPT - 7 * ZC
            pltpu.sync_copy(acc_sh.at[pl.ds(tbase + q * ZC, n)],
                            ones_v.at[pl.ds(0, n)])
            pltpu.sync_copy(ones_v.at[pl.ds(0, n)],
                            out_hbm.at[pl.ds(c * N + tbase + q * ZC, n)])

        @pl.when(s == NT - 1)
        def _():
            pltpu.sync_copy(acc_sh.at[pl.ds(NT * RPT, 16)],
                            zbuf.at[pl.ds(0, 16)])
            pltpu.sync_copy(zbuf.at[pl.ds(0, 16)],
                            out_hbm.at[pl.ds(c * N + NT * RPT, 16)])

    return deg_kernel(dst3)


def _sc_mp(hp2, src4, dst3):
    @functools.partial(
        pl.kernel,
        mesh=_mesh(),
        out_type=jax.ShapeDtypeStruct((2 * N, H), jnp.float32),
        compiler_params=pltpu.CompilerParams(
            use_tc_tiling_on_sc=False, skip_device_barrier=True),
        scratch_types=[
            pltpu.VMEM((KPT, C), jnp.int32),
            pltpu.VMEM((KPT, C), jnp.int32),
            pltpu.VMEM((PB, C, H), jnp.float32),
            pltpu.VMEM_SHARED((N, H), jnp.float32),
            [pltpu.SemaphoreType.DMA] * PB,
            [pltpu.SemaphoreType.DMA] * PB,
        ],
    )
    def mp_kernel(hp_hbm, src_hbm, dst_hbm, out_hbm,
                  src_v, dst_v, gbuf, acc_sh, gsems, ssems):
        c = lax.axis_index("c")
        s = lax.axis_index("s")

        pltpu.sync_copy(src_hbm.at[c, s], src_v)
        pltpu.sync_copy(dst_hbm.at[s], dst_v)

        def zrow(i, carry):
            for k in range(H // 16):
                gbuf[0, i, pl.ds(k * 16, 16)] = jnp.zeros((16,), jnp.float32)
            return carry

        lax.fori_loop(0, ZC, zrow, 0)
        tbase = s * RPT
        for q in range(8):
            n = ZC if q < 7 else RPT - 7 * ZC
            pltpu.sync_copy(gbuf.at[0].at[pl.ds(0, n)],
                            acc_sh.at[pl.ds(tbase + q * ZC, n)])

        @pl.when(s == NT - 1)
        def _():
            pltpu.sync_copy(gbuf.at[0].at[pl.ds(0, 16)],
                            acc_sh.at[pl.ds(NT * RPT, 16)])

        plsc.subcore_barrier()

        def gstart(j, b):
            pltpu.async_copy(hp_hbm.at[src_v.at[j]], gbuf.at[b], gsems[b])

        def gwait(j, b):
            pltpu.make_async_copy(hp_hbm.at[src_v.at[j]], gbuf.at[b],
                                  gsems[b]).wait()

        def sstart(j, b):
            pltpu.async_copy(gbuf.at[b], acc_sh.at[dst_v.at[j]], ssems[b],
                             add=True)

        def swait(j, b):
            pltpu.make_async_copy(gbuf.at[b], acc_sh.at[dst_v.at[j]],
                                  ssems[b]).wait()

        for j in range(GA):
            gstart(j, j)
        for j in range(GA):
            gwait(j, j)
            sstart(j, j)
            gstart(j + GA, (j + GA) % PB)

        NK = (KPT - GA - PB + 1) // PB

        def body(k, carry):
            j0 = PB * k + GA
            for t in range(PB):
                j = j0 + t
                b = (GA + t) % PB
                bb = (2 * GA + t) % PB
                gwait(j, b)
                sstart(j, b)
                swait(j + GA - PB, bb)
                gstart(j + GA, bb)
            return carry

        lax.fori_loop(0, NK, body, 0)
        for j in range(PB * NK + GA, KPT):
            b = j % PB
            bb = (j + GA) % PB
            gwait(j, b)
            sstart(j, b)
            swait(j + GA - PB, bb)
            if j + GA < KPT:
                gstart(j + GA, bb)
        for j in range(KPT - (PB - GA), KPT):
            swait(j, j % PB)

        plsc.subcore_barrier()

        for q in range(8):
            n = ZC if q < 7 else RPT - 7 * ZC
            pltpu.sync_copy(acc_sh.at[pl.ds(tbase + q * ZC, n)],
                            gbuf.at[0].at[pl.ds(0, n)])
            pltpu.sync_copy(gbuf.at[0].at[pl.ds(0, n)],
                            out_hbm.at[pl.ds(c * N + tbase + q * ZC, n)])

        @pl.when(s == NT - 1)
        def _():
            pltpu.sync_copy(acc_sh.at[pl.ds(NT * RPT, 16)],
                            gbuf.at[1].at[pl.ds(0, 16)])
            pltpu.sync_copy(gbuf.at[1].at[pl.ds(0, 16)],
                            out_hbm.at[pl.ds(c * N + NT * RPT, 16)])

    return mp_kernel(hp2, src4, dst3)


def _tc_first(degp3, x, Wsp):
    def body(deg_ref, x_ref, w_ref, o_ref):
        deg = (jnp.sum(deg_ref[0], axis=1, keepdims=True)
               + jnp.sum(deg_ref[1], axis=1, keepdims=True) + 1.0)
        dis = lax.rsqrt(deg)
        h = jnp.dot(x_ref[...], w_ref[0], preferred_element_type=jnp.float32)
        o_ref[0] = h * dis

    return pl.pallas_call(
        body,
        grid=(2, N // BM),
        in_specs=[
            pl.BlockSpec((2, BM, DW), lambda h, i: (0, i, 0)),
            pl.BlockSpec((BM, D), lambda h, i: (i, 0)),
            pl.BlockSpec((1, D, H), lambda h, i: (h, 0, 0)),
        ],
        out_specs=pl.BlockSpec((1, BM, H), lambda h, i: (h, i, 0)),
        out_shape=jax.ShapeDtypeStruct((2, N, H), jnp.float32),
    )(degp3, x, Wsp)


def _tc_mid(degp3, acc, hp, Wq, bsp):
    def body(deg_ref, a_ref, hp_ref, w_ref, b_ref, o_ref):
        deg = (jnp.sum(deg_ref[0], axis=1, keepdims=True)
               + jnp.sum(deg_ref[1], axis=1, keepdims=True) + 1.0)
        dis = lax.rsqrt(deg)
        x0 = jnp.maximum((a_ref[0] + hp_ref[0]) * dis + b_ref[0], 0.0)
        x1 = jnp.maximum((a_ref[1] + hp_ref[1]) * dis + b_ref[1], 0.0)
        h = (jnp.dot(x0, w_ref[0, 0], preferred_element_type=jnp.float32)
             + jnp.dot(x1, w_ref[1, 0], preferred_element_type=jnp.float32))
        o_ref[0] = h * dis

    return pl.pallas_call(
        body,
        grid=(2, N // BM),
        in_specs=[
            pl.BlockSpec((2, BM, DW), lambda h, i: (0, i, 0)),
            pl.BlockSpec((2, BM, H), lambda h, i: (0, i, 0)),
            pl.BlockSpec((2, BM, H), lambda h, i: (0, i, 0)),
            pl.BlockSpec((2, 1, H, H), lambda h, i: (0, h, 0, 0)),
            pl.BlockSpec((2, 1, H), lambda h, i: (0, 0, 0)),
        ],
        out_specs=pl.BlockSpec((1, BM, H), lambda h, i: (h, i, 0)),
        out_shape=jax.ShapeDtypeStruct((2, N, H), jnp.float32),
    )(degp3, acc, hp, Wq, bsp)


def _tc_last(degp3, acc, hp, b):
    def body(deg_ref, a_ref, hp_ref, b_ref, o_ref):
        deg = (jnp.sum(deg_ref[0], axis=1, keepdims=True)
               + jnp.sum(deg_ref[1], axis=1, keepdims=True) + 1.0)
        dis = lax.rsqrt(deg)
        y0 = (a_ref[0] + hp_ref[0]) * dis
        y1 = (a_ref[1] + hp_ref[1]) * dis
        o_ref[...] = jnp.concatenate([y0, y1], axis=1) + b_ref[...]

    return pl.pallas_call(
        body,
        grid=(N // BM,),
        in_specs=[
            pl.BlockSpec((2, BM, DW), lambda i: (0, i, 0)),
            pl.BlockSpec((2, BM, H), lambda i: (0, i, 0)),
            pl.BlockSpec((2, BM, H), lambda i: (0, i, 0)),
            pl.BlockSpec((1, D), lambda i: (0, 0)),
        ],
        out_specs=pl.BlockSpec((BM, D), lambda i: (i, 0)),
        out_shape=jax.ShapeDtypeStruct((N, D), jnp.float32),
    )(degp3, acc, hp, b)


def kernel(x, adj_t, W1, b1, W2, b2, W3, b3):
    adj = adj_t.astype(jnp.int32)
    src3 = adj[0].reshape(NT, KPT, C)
    src4 = jnp.stack([src3, src3 + N])
    dst3 = adj[1].reshape(NT, KPT, C)
    dst3d = adj[1].reshape(NW, KPW, C)

    degp = _sc_degree(dst3d)
    degp3 = degp.reshape(2, N, DW)

    def wq(W):
        return W.reshape(2, H, 2, H).transpose(0, 2, 1, 3)

    def wsp(W):
        return W.reshape(D, 2, H).transpose(1, 0, 2)

    hp1 = _tc_first(degp3, x, wsp(W1))
    acc1 = _sc_mp(hp1.reshape(2 * N, H), src4, dst3).reshape(2, N, H)
    hp2 = _tc_mid(degp3, acc1, hp1, wq(W2), b1.reshape(2, 1, H))
    acc2 = _sc_mp(hp2.reshape(2 * N, H), src4, dst3).reshape(2, N, H)
    hp3 = _tc_mid(degp3, acc2, hp2, wq(W3), b2.reshape(2, 1, H))
    acc3 = _sc_mp(hp3.reshape(2 * N, H), src4, dst3).reshape(2, N, H)
    return _tc_last(degp3, acc3, hp3, b3.reshape(1, D))


# speedup vs baseline: 1.0664x; 1.0110x over previous
import functools

import jax
import jax.numpy as jnp
from jax import lax
from jax.experimental import pallas as pl
from jax.experimental.pallas import tpu as pltpu
from jax.experimental.pallas import tpu_sc as plsc

N = 10000
E = 320000
D = 128
H = 64
C = 125
ZC = 80
NT = 16
EPT = E // NT
KPT = EPT // C
NW = 32
EPW = E // NW
KPW = EPW // C
RPT = 624
PB = 6
GA = 3
BM = 10000


def _mesh():
    return plsc.VectorSubcoreMesh(core_axis_name="c", subcore_axis_name="s")


DW = 16


def _sc_degree(adjd):
    @functools.partial(
        pl.kernel,
        mesh=_mesh(),
        out_type=jax.ShapeDtypeStruct((2, N, DW), jnp.float32),
        compiler_params=pltpu.CompilerParams(
            use_tc_tiling_on_sc=False, skip_device_barrier=True),
        scratch_types=[
            pltpu.VMEM((KPW, C), jnp.int32),
            pltpu.VMEM((C, DW), jnp.float32),
            pltpu.VMEM((C, DW), jnp.float32),
            pltpu.VMEM_SHARED((N, DW), jnp.float32),
        ],
    )
    def deg_kernel(dst_hbm, out_hbm, dst_v, ones_v, zbuf, acc_sh):
        c = lax.axis_index("c")
        s = lax.axis_index("s")
        wid = s * 2 + c

        def fill(i, carry):
            ones_v[i, pl.ds(0, 16)] = jnp.full((16,), 1.0 / DW, jnp.float32)
            zbuf[i, pl.ds(0, 16)] = jnp.zeros((16,), jnp.float32)
            return carry

        lax.fori_loop(0, C, fill, 0)
        pltpu.sync_copy(dst_hbm.at[1, wid], dst_v)

        tbase = s * RPT
        for q in range(8):
            n = ZC if q < 7 else RPT - 7 * ZC
            pltpu.sync_copy(zbuf.at[pl.ds(0, n)],
                            acc_sh.at[pl.ds(tbase + q * ZC, n)])

        @pl.when(s == NT - 1)
        def _():
            pltpu.sync_copy(zbuf.at[pl.ds(0, 16)],
                            acc_sh.at[pl.ds(NT * RPT, 16)])

        plsc.subcore_barrier()

        def body(j, carry):
            pltpu.sync_copy(ones_v, acc_sh.at[dst_v.at[j]], add=True)
            return carry

        lax.fori_loop(0, KPW, body, 0)
        plsc.subcore_barrier()

        for q in range(8):
            n = ZC if q < 7 else RPT - 7 * ZC
            pltpu.sync_copy(acc_sh.at[pl.ds(tbase + q * ZC, n)],
                            ones_v.at[pl.ds(0, n)])
            pltpu.sync_copy(ones_v.at[pl.ds(0, n)],
                            out_hbm.at[c, pl.ds(tbase + q * ZC, n)])

        @pl.when(s == NT - 1)
        def _():
            pltpu.sync_copy(acc_sh.at[pl.ds(NT * RPT, 16)],
                            zbuf.at[pl.ds(0, 16)])
            pltpu.sync_copy(zbuf.at[pl.ds(0, 16)],
                            out_hbm.at[c, pl.ds(NT * RPT, 16)])

    return deg_kernel(adjd)


def _sc_mp(hp3, adjm):
    @functools.partial(
        pl.kernel,
        mesh=_mesh(),
        out_type=jax.ShapeDtypeStruct((2, N, H), jnp.float32),
        compiler_params=pltpu.CompilerParams(
            use_tc_tiling_on_sc=False, skip_device_barrier=True),
        scratch_types=[
            pltpu.VMEM((KPT, C), jnp.int32),
            pltpu.VMEM((KPT, C), jnp.int32),
            pltpu.VMEM((PB, C, H), jnp.float32),
            pltpu.VMEM_SHARED((N, H), jnp.float32),
            [pltpu.SemaphoreType.DMA] * PB,
            [pltpu.SemaphoreType.DMA] * PB,
        ],
    )
    def mp_kernel(hp_hbm, adj_hbm, out_hbm,
                  src_v, dst_v, gbuf, acc_sh, gsems, ssems):
        c = lax.axis_index("c")
        s = lax.axis_index("s")
        hp_c = hp_hbm.at[c]

        pltpu.sync_copy(adj_hbm.at[0, s], src_v)
        pltpu.sync_copy(adj_hbm.at[1, s], dst_v)

        def zrow(i, carry):
            for k in range(H // 16):
                gbuf[0, i, pl.ds(k * 16, 16)] = jnp.zeros((16,), jnp.float32)
            return carry

        lax.fori_loop(0, ZC, zrow, 0)
        tbase = s * RPT
        for q in range(8):
            n = ZC if q < 7 else RPT - 7 * ZC
            pltpu.sync_copy(gbuf.at[0].at[pl.ds(0, n)],
                            acc_sh.at[pl.ds(tbase + q * ZC, n)])

        @pl.when(s == NT - 1)
        def _():
            pltpu.sync_copy(gbuf.at[0].at[pl.ds(0, 16)],
                            acc_sh.at[pl.ds(NT * RPT, 16)])

        plsc.subcore_barrier()

        def gstart(j, b):
            pltpu.async_copy(hp_c.at[src_v.at[j]], gbuf.at[b], gsems[b])

        def gwait(j, b):
            pltpu.make_async_copy(hp_c.at[src_v.at[j]], gbuf.at[b],
                                  gsems[b]).wait()

        def sstart(j, b):
            pltpu.async_copy(gbuf.at[b], acc_sh.at[dst_v.at[j]], ssems[b],
                             add=True)

        def swait(j, b):
            pltpu.make_async_copy(gbuf.at[b], acc_sh.at[dst_v.at[j]],
                                  ssems[b]).wait()

        for j in range(GA):
            gstart(j, j)
        for j in range(GA):
            gwait(j, j)
            sstart(j, j)
            gstart(j + GA, (j + GA) % PB)

        NK = (KPT - GA - PB + 1) // PB

        def body(k, carry):
            j0 = PB * k + GA
            for t in range(PB):
                j = j0 + t
                b = (GA + t) % PB
                bb = (2 * GA + t) % PB
                gwait(j, b)
                sstart(j, b)
                swait(j + GA - PB, bb)
                gstart(j + GA, bb)
            return carry

        lax.fori_loop(0, NK, body, 0)
        for j in range(PB * NK + GA, KPT):
            b = j % PB
            bb = (j + GA) % PB
            gwait(j, b)
            sstart(j, b)
            swait(j + GA - PB, bb)
            if j + GA < KPT:
                gstart(j + GA, bb)
        for j in range(KPT - (PB - GA), KPT):
            swait(j, j % PB)

        plsc.subcore_barrier()

        for q in range(8):
            n = ZC if q < 7 else RPT - 7 * ZC
            pltpu.sync_copy(acc_sh.at[pl.ds(tbase + q * ZC, n)],
                            gbuf.at[0].at[pl.ds(0, n)])
            pltpu.sync_copy(gbuf.at[0].at[pl.ds(0, n)],
                            out_hbm.at[c, pl.ds(tbase + q * ZC, n)])

        @pl.when(s == NT - 1)
        def _():
            pltpu.sync_copy(acc_sh.at[pl.ds(NT * RPT, 16)],
                            gbuf.at[1].at[pl.ds(0, 16)])
            pltpu.sync_copy(gbuf.at[1].at[pl.ds(0, 16)],
                            out_hbm.at[c, pl.ds(NT * RPT, 16)])

    return mp_kernel(hp3, adjm)


def _tc_first(degp3, x, Wsp):
    def body(deg_ref, x_ref, w_ref, o_ref):
        deg = (jnp.sum(deg_ref[0], axis=1, keepdims=True)
               + jnp.sum(deg_ref[1], axis=1, keepdims=True) + 1.0)
        dis = lax.rsqrt(deg)
        h = jnp.dot(x_ref[...], w_ref[0], preferred_element_type=jnp.float32)
        o_ref[0] = h * dis

    return pl.pallas_call(
        body,
        grid=(2, N // BM),
        in_specs=[
            pl.BlockSpec((2, BM, DW), lambda h, i: (0, i, 0)),
            pl.BlockSpec((BM, D), lambda h, i: (i, 0)),
            pl.BlockSpec((1, D, H), lambda h, i: (h, 0, 0)),
        ],
        out_specs=pl.BlockSpec((1, BM, H), lambda h, i: (h, i, 0)),
        out_shape=jax.ShapeDtypeStruct((2, N, H), jnp.float32),
    )(degp3, x, Wsp)


def _tc_mid(degp3, acc, hp, Wq, bsp):
    def body(deg_ref, a_ref, hp_ref, w_ref, b_ref, o_ref):
        deg = (jnp.sum(deg_ref[0], axis=1, keepdims=True)
               + jnp.sum(deg_ref[1], axis=1, keepdims=True) + 1.0)
        dis = lax.rsqrt(deg)
        x0 = jnp.maximum((a_ref[0] + hp_ref[0]) * dis + b_ref[0], 0.0)
        x1 = jnp.maximum((a_ref[1] + hp_ref[1]) * dis + b_ref[1], 0.0)
        h = (jnp.dot(x0, w_ref[0, 0], preferred_element_type=jnp.float32)
             + jnp.dot(x1, w_ref[1, 0], preferred_element_type=jnp.float32))
        o_ref[0] = h * dis

    return pl.pallas_call(
        body,
        grid=(2, N // BM),
        in_specs=[
            pl.BlockSpec((2, BM, DW), lambda h, i: (0, i, 0)),
            pl.BlockSpec((2, BM, H), lambda h, i: (0, i, 0)),
            pl.BlockSpec((2, BM, H), lambda h, i: (0, i, 0)),
            pl.BlockSpec((2, 1, H, H), lambda h, i: (0, h, 0, 0)),
            pl.BlockSpec((2, 1, H), lambda h, i: (0, 0, 0)),
        ],
        out_specs=pl.BlockSpec((1, BM, H), lambda h, i: (h, i, 0)),
        out_shape=jax.ShapeDtypeStruct((2, N, H), jnp.float32),
    )(degp3, acc, hp, Wq, bsp)


def _tc_last(degp3, acc, hp, b):
    def body(deg_ref, a_ref, hp_ref, b_ref, o_ref):
        deg = (jnp.sum(deg_ref[0], axis=1, keepdims=True)
               + jnp.sum(deg_ref[1], axis=1, keepdims=True) + 1.0)
        dis = lax.rsqrt(deg)
        y0 = (a_ref[0] + hp_ref[0]) * dis
        y1 = (a_ref[1] + hp_ref[1]) * dis
        o_ref[...] = jnp.concatenate([y0, y1], axis=1) + b_ref[...]

    return pl.pallas_call(
        body,
        grid=(N // BM,),
        in_specs=[
            pl.BlockSpec((2, BM, DW), lambda i: (0, i, 0)),
            pl.BlockSpec((2, BM, H), lambda i: (0, i, 0)),
            pl.BlockSpec((2, BM, H), lambda i: (0, i, 0)),
            pl.BlockSpec((1, D), lambda i: (0, 0)),
        ],
        out_specs=pl.BlockSpec((BM, D), lambda i: (i, 0)),
        out_shape=jax.ShapeDtypeStruct((N, D), jnp.float32),
    )(degp3, acc, hp, b)


def kernel(x, adj_t, W1, b1, W2, b2, W3, b3):
    adj = adj_t.astype(jnp.int32)
    adjd = adj.reshape(2, NW, KPW, C)
    adjm = adj.reshape(2, NT, KPT, C)

    degp3 = _sc_degree(adjd)

    def wq(W):
        return W.reshape(2, H, 2, H).transpose(0, 2, 1, 3)

    def wsp(W):
        return W.reshape(D, 2, H).transpose(1, 0, 2)

    hp1 = _tc_first(degp3, x, wsp(W1))
    acc1 = _sc_mp(hp1, adjm)
    hp2 = _tc_mid(degp3, acc1, hp1, wq(W2), b1.reshape(2, 1, H))
    acc2 = _sc_mp(hp2, adjm)
    hp3 = _tc_mid(degp3, acc2, hp2, wq(W3), b2.reshape(2, 1, H))
    acc3 = _sc_mp(hp3, adjm)
    return _tc_last(degp3, acc3, hp3, b3.reshape(1, D))


# speedup vs baseline: 1.0780x; 1.0109x over previous
import functools

import jax
import jax.numpy as jnp
from jax import lax
from jax.experimental import pallas as pl
from jax.experimental.pallas import tpu as pltpu
from jax.experimental.pallas import tpu_sc as plsc

N = 10000
E = 320000
D = 128
H = 64
C = 125
ZC = 80
NT = 16
EPT = E // NT
KPT = EPT // C
NW = 32
EPW = E // NW
KPW = EPW // C
RPT = 624
PB = 6
GA = 3
BM = 10000


def _mesh():
    return plsc.VectorSubcoreMesh(core_axis_name="c", subcore_axis_name="s")


DW = 16


def _sc_degree(adjm):
    @functools.partial(
        pl.kernel,
        mesh=_mesh(),
        out_type=jax.ShapeDtypeStruct((2, N, DW), jnp.float32),
        compiler_params=pltpu.CompilerParams(
            use_tc_tiling_on_sc=False, skip_device_barrier=True),
        scratch_types=[
            pltpu.VMEM((KPW, C), jnp.int32),
            pltpu.VMEM((C, DW), jnp.float32),
            pltpu.VMEM((C, DW), jnp.float32),
            pltpu.VMEM_SHARED((N, DW), jnp.float32),
        ],
    )
    def deg_kernel(dst_hbm, out_hbm, dst_v, ones_v, zbuf, acc_sh):
        c = lax.axis_index("c")
        s = lax.axis_index("s")

        def fill(i, carry):
            ones_v[i, pl.ds(0, 16)] = jnp.full((16,), 1.0 / DW, jnp.float32)
            zbuf[i, pl.ds(0, 16)] = jnp.zeros((16,), jnp.float32)
            return carry

        lax.fori_loop(0, C, fill, 0)
        pltpu.sync_copy(dst_hbm.at[1, s, pl.ds(c * KPW, KPW)], dst_v)

        tbase = s * RPT
        for q in range(8):
            n = ZC if q < 7 else RPT - 7 * ZC
            pltpu.sync_copy(zbuf.at[pl.ds(0, n)],
                            acc_sh.at[pl.ds(tbase + q * ZC, n)])

        @pl.when(s == NT - 1)
        def _():
            pltpu.sync_copy(zbuf.at[pl.ds(0, 16)],
                            acc_sh.at[pl.ds(NT * RPT, 16)])

        plsc.subcore_barrier()

        def body(j, carry):
            pltpu.sync_copy(ones_v, acc_sh.at[dst_v.at[j]], add=True)
            return carry

        lax.fori_loop(0, KPW, body, 0)
        plsc.subcore_barrier()

        for q in range(8):
            n = ZC if q < 7 else RPT - 7 * ZC
            pltpu.sync_copy(acc_sh.at[pl.ds(tbase + q * ZC, n)],
                            ones_v.at[pl.ds(0, n)])
            pltpu.sync_copy(ones_v.at[pl.ds(0, n)],
                            out_hbm.at[c, pl.ds(tbase + q * ZC, n)])

        @pl.when(s == NT - 1)
        def _():
            pltpu.sync_copy(acc_sh.at[pl.ds(NT * RPT, 16)],
                            zbuf.at[pl.ds(0, 16)])
            pltpu.sync_copy(zbuf.at[pl.ds(0, 16)],
                            out_hbm.at[c, pl.ds(NT * RPT, 16)])

    return deg_kernel(adjm)


def _sc_mp(hp3, adjm):
    @functools.partial(
        pl.kernel,
        mesh=_mesh(),
        out_type=jax.ShapeDtypeStruct((2, N, H), jnp.float32),
        compiler_params=pltpu.CompilerParams(
            use_tc_tiling_on_sc=False, skip_device_barrier=True),
        scratch_types=[
            pltpu.VMEM((KPT, C), jnp.int32),
            pltpu.VMEM((KPT, C), jnp.int32),
            pltpu.VMEM((PB, C, H), jnp.float32),
            pltpu.VMEM_SHARED((N, H), jnp.float32),
            [pltpu.SemaphoreType.DMA] * PB,
            [pltpu.SemaphoreType.DMA] * PB,
        ],
    )
    def mp_kernel(hp_hbm, adj_hbm, out_hbm,
                  src_v, dst_v, gbuf, acc_sh, gsems, ssems):
        c = lax.axis_index("c")
        s = lax.axis_index("s")
        hp_c = hp_hbm.at[c]

        pltpu.sync_copy(adj_hbm.at[0, s], src_v)
        pltpu.sync_copy(adj_hbm.at[1, s], dst_v)

        def zrow(i, carry):
            for k in range(H // 16):
                gbuf[0, i, pl.ds(k * 16, 16)] = jnp.zeros((16,), jnp.float32)
            return carry

        lax.fori_loop(0, ZC, zrow, 0)
        tbase = s * RPT
        for q in range(8):
            n = ZC if q < 7 else RPT - 7 * ZC
            pltpu.sync_copy(gbuf.at[0].at[pl.ds(0, n)],
                            acc_sh.at[pl.ds(tbase + q * ZC, n)])

        @pl.when(s == NT - 1)
        def _():
            pltpu.sync_copy(gbuf.at[0].at[pl.ds(0, 16)],
                            acc_sh.at[pl.ds(NT * RPT, 16)])

        plsc.subcore_barrier()

        def gstart(j, b):
            pltpu.async_copy(hp_c.at[src_v.at[j]], gbuf.at[b], gsems[b])

        def gwait(j, b):
            pltpu.make_async_copy(hp_c.at[src_v.at[j]], gbuf.at[b],
                                  gsems[b]).wait()

        def sstart(j, b):
            pltpu.async_copy(gbuf.at[b], acc_sh.at[dst_v.at[j]], ssems[b],
                             add=True)

        def swait(j, b):
            pltpu.make_async_copy(gbuf.at[b], acc_sh.at[dst_v.at[j]],
                                  ssems[b]).wait()

        for j in range(GA):
            gstart(j, j)
        for j in range(GA):
            gwait(j, j)
            sstart(j, j)
            gstart(j + GA, (j + GA) % PB)

        NK = (KPT - GA - PB + 1) // PB

        def body(k, carry):
            j0 = PB * k + GA
            for t in range(PB):
                j = j0 + t
                b = (GA + t) % PB
                bb = (2 * GA + t) % PB
                gwait(j, b)
                sstart(j, b)
                swait(j + GA - PB, bb)
                gstart(j + GA, bb)
            return carry

        lax.fori_loop(0, NK, body, 0)
        for j in range(PB * NK + GA, KPT):
            b = j % PB
            bb = (j + GA) % PB
            gwait(j, b)
            sstart(j, b)
            swait(j + GA - PB, bb)
            if j + GA < KPT:
                gstart(j + GA, bb)
        for j in range(KPT - (PB - GA), KPT):
            swait(j, j % PB)

        plsc.subcore_barrier()

        for q in range(8):
            n = ZC if q < 7 else RPT - 7 * ZC
            pltpu.sync_copy(acc_sh.at[pl.ds(tbase + q * ZC, n)],
                            gbuf.at[0].at[pl.ds(0, n)])
            pltpu.sync_copy(gbuf.at[0].at[pl.ds(0, n)],
                            out_hbm.at[c, pl.ds(tbase + q * ZC, n)])

        @pl.when(s == NT - 1)
        def _():
            pltpu.sync_copy(acc_sh.at[pl.ds(NT * RPT, 16)],
                            gbuf.at[1].at[pl.ds(0, 16)])
            pltpu.sync_copy(gbuf.at[1].at[pl.ds(0, 16)],
                            out_hbm.at[c, pl.ds(NT * RPT, 16)])

    return mp_kernel(hp3, adjm)


def _tc_first(degp3, x, Wsp):
    def body(deg_ref, x_ref, w_ref, o_ref):
        deg = (jnp.sum(deg_ref[0], axis=1, keepdims=True)
               + jnp.sum(deg_ref[1], axis=1, keepdims=True) + 1.0)
        dis = lax.rsqrt(deg)
        o_ref[0] = jnp.dot(x_ref[...], w_ref[0],
                           preferred_element_type=jnp.float32) * dis
        o_ref[1] = jnp.dot(x_ref[...], w_ref[1],
                           preferred_element_type=jnp.float32) * dis

    return pl.pallas_call(
        body,
        grid=(1,),
        in_specs=[
            pl.BlockSpec((2, N, DW), lambda i: (0, 0, 0)),
            pl.BlockSpec((N, D), lambda i: (0, 0)),
            pl.BlockSpec((2, D, H), lambda i: (0, 0, 0)),
        ],
        out_specs=pl.BlockSpec((2, N, H), lambda i: (0, 0, 0)),
        out_shape=jax.ShapeDtypeStruct((2, N, H), jnp.float32),
    )(degp3, x, Wsp)


def _tc_mid(degp3, acc, hp, Wq, bsp):
    def body(deg_ref, a_ref, hp_ref, w_ref, b_ref, o_ref):
        deg = (jnp.sum(deg_ref[0], axis=1, keepdims=True)
               + jnp.sum(deg_ref[1], axis=1, keepdims=True) + 1.0)
        dis = lax.rsqrt(deg)
        x0 = jnp.maximum((a_ref[0] + hp_ref[0]) * dis + b_ref[0], 0.0)
        x1 = jnp.maximum((a_ref[1] + hp_ref[1]) * dis + b_ref[1], 0.0)
        o_ref[0] = (jnp.dot(x0, w_ref[0, 0], preferred_element_type=jnp.float32)
                    + jnp.dot(x1, w_ref[1, 0],
                              preferred_element_type=jnp.float32)) * dis
        o_ref[1] = (jnp.dot(x0, w_ref[0, 1], preferred_element_type=jnp.float32)
                    + jnp.dot(x1, w_ref[1, 1],
                              preferred_element_type=jnp.float32)) * dis

    return pl.pallas_call(
        body,
        grid=(1,),
        in_specs=[
            pl.BlockSpec((2, N, DW), lambda i: (0, 0, 0)),
            pl.BlockSpec((2, N, H), lambda i: (0, 0, 0)),
            pl.BlockSpec((2, N, H), lambda i: (0, 0, 0)),
            pl.BlockSpec((2, 2, H, H), lambda i: (0, 0, 0, 0)),
            pl.BlockSpec((2, 1, H), lambda i: (0, 0, 0)),
        ],
        out_specs=pl.BlockSpec((2, N, H), lambda i: (0, 0, 0)),
        out_shape=jax.ShapeDtypeStruct((2, N, H), jnp.float32),
    )(degp3, acc, hp, Wq, bsp)


def _tc_last(degp3, acc, hp, b):
    def body(deg_ref, a_ref, hp_ref, b_ref, o_ref):
        deg = (jnp.sum(deg_ref[0], axis=1, keepdims=True)
               + jnp.sum(deg_ref[1], axis=1, keepdims=True) + 1.0)
        dis = lax.rsqrt(deg)
        y0 = (a_ref[0] + hp_ref[0]) * dis
        y1 = (a_ref[1] + hp_ref[1]) * dis
        o_ref[...] = jnp.concatenate([y0, y1], axis=1) + b_ref[...]

    return pl.pallas_call(
        body,
        grid=(1,),
        in_specs=[
            pl.BlockSpec((2, N, DW), lambda i: (0, 0, 0)),
            pl.BlockSpec((2, N, H), lambda i: (0, 0, 0)),
            pl.BlockSpec((2, N, H), lambda i: (0, 0, 0)),
            pl.BlockSpec((1, D), lambda i: (0, 0)),
        ],
        out_specs=pl.BlockSpec((N, D), lambda i: (0, 0)),
        out_shape=jax.ShapeDtypeStruct((N, D), jnp.float32),
    )(degp3, acc, hp, b)


def kernel(x, adj_t, W1, b1, W2, b2, W3, b3):
    adj = adj_t.astype(jnp.int32)
    adjm = adj.reshape(2, NT, KPT, C)

    degp3 = _sc_degree(adjm)

    def wq(W):
        return W.reshape(2, H, 2, H).transpose(0, 2, 1, 3)

    def wsp(W):
        return W.reshape(D, 2, H).transpose(1, 0, 2)

    hp1 = _tc_first(degp3, x, wsp(W1))
    acc1 = _sc_mp(hp1, adjm)
    hp2 = _tc_mid(degp3, acc1, hp1, wq(W2), b1.reshape(2, 1, H))
    acc2 = _sc_mp(hp2, adjm)
    hp3 = _tc_mid(degp3, acc2, hp2, wq(W3), b2.reshape(2, 1, H))
    acc3 = _sc_mp(hp3, adjm)
    return _tc_last(degp3, acc3, hp3, b3.reshape(1, D))
